# Initial kernel scaffold; baseline (speedup 1.0000x reference)
import jax
import jax.numpy as jnp
from jax.experimental import pallas as pl


def kernel(x, adj, params, edge_index):
    raise NotImplementedError("write your pallas kernel here")


# speedup vs baseline: 1.0705x; 1.0705x over previous
import functools

import jax
import jax.numpy as jnp
from jax import lax
from jax.experimental import pallas as pl

N = 10000
N_Z = 20
N_CLUSTERS = 10
A = 0.5
ALPHA = 0.45
BETA = 0.25

_MM_ROWS = 2000
_BIG_ROWS = 200


def _mm_body(h_ref, w_ref, b_ref, o_ref, *, act):
    o = jnp.dot(h_ref[...], w_ref[...], preferred_element_type=jnp.float32)
    o = o + b_ref[...]
    if act:
        o = jnp.maximum(o, 0.0)
    o_ref[...] = o


def _dense(h, w, b, act):
    n, din = h.shape
    dout = w.shape[1]
    if b is None:
        b = jnp.zeros((dout,), jnp.float32)
    b2 = b.reshape(1, dout)
    return pl.pallas_call(
        functools.partial(_mm_body, act=act),
        grid=(n // _MM_ROWS,),
        in_specs=[
            pl.BlockSpec((_MM_ROWS, din), lambda i: (i, 0)),
            pl.BlockSpec((din, dout), lambda i: (0, 0)),
            pl.BlockSpec((1, dout), lambda i: (0, 0)),
        ],
        out_specs=pl.BlockSpec((_MM_ROWS, dout), lambda i: (i, 0)),
        out_shape=jax.ShapeDtypeStruct((n, dout), jnp.float32),
    )(h, w, b2)


def _tdist_body(z_ref, c_ref, o_ref):
    z = z_ref[...]
    c = c_ref[...]
    d2 = (
        jnp.sum(z * z, axis=1, keepdims=True)
        - 2.0 * lax.dot_general(z, c, (((1,), (1,)), ((), ())),
                                preferred_element_type=jnp.float32)
        + jnp.sum(c * c, axis=1)[None, :]
    )
    u = 1.0 / (1.0 + d2)
    o_ref[...] = u / jnp.sum(u, axis=1, keepdims=True)


def _tdist(z, cluster):
    n = z.shape[0]
    k, dz = cluster.shape
    return pl.pallas_call(
        _tdist_body,
        grid=(n // _MM_ROWS,),
        in_specs=[
            pl.BlockSpec((_MM_ROWS, dz), lambda i: (i, 0)),
            pl.BlockSpec((k, dz), lambda i: (0, 0)),
        ],
        out_specs=pl.BlockSpec((_MM_ROWS, k), lambda i: (i, 0)),
        out_shape=jax.ShapeDtypeStruct((n, k), jnp.float32),
    )(z, cluster)


def _big_body(adj_ref, zit_ref, zgb_ref, zgt_ref, zl_ref, eg_ref):
    adj = adj_ref[...]
    zit = zit_ref[...]
    zgt = zgt_ref[...]
    zl_ref[...] = lax.dot_general(
        adj, zit, (((1,), (1,)), ((), ())), preferred_element_type=jnp.float32)
    zgb = zgb_ref[...]
    s = jnp.dot(zgb, zgt, preferred_element_type=jnp.float32)
    eg_ref[...] = jax.nn.sigmoid(s)


def _big(adj, z_i, z_gcn):
    n = adj.shape[0]
    zit = z_i.T
    zgt = z_gcn.T
    return pl.pallas_call(
        _big_body,
        grid=(n // _BIG_ROWS,),
        in_specs=[
            pl.BlockSpec((_BIG_ROWS, n), lambda i: (i, 0)),
            pl.BlockSpec((N_Z, n), lambda i: (0, 0)),
            pl.BlockSpec((_BIG_ROWS, N_Z), lambda i: (i, 0)),
            pl.BlockSpec((N_Z, n), lambda i: (0, 0)),
        ],
        out_specs=[
            pl.BlockSpec((_BIG_ROWS, N_Z), lambda i: (i, 0)),
            pl.BlockSpec((_BIG_ROWS, n), lambda i: (i, 0)),
        ],
        out_shape=[
            jax.ShapeDtypeStruct((n, N_Z), jnp.float32),
            jax.ShapeDtypeStruct((n, n), jnp.float32),
        ],
    )(adj, zit, z_gcn, zgt)


def kernel(x, adj, params, edge_index):
    p = params
    n = x.shape[0]
    src = edge_index[0].astype(jnp.int32)
    dst = edge_index[1].astype(jnp.int32)
    relu = jax.nn.relu

    e1 = _dense(x, p['We1'], p['be1'], True)
    e2 = _dense(e1, p['We2'], p['be2'], True)
    e3 = _dense(e2, p['We3'], p['be3'], True)
    z_ae = _dense(e3, p['Wz'], p['bz'], False)
    d1 = _dense(z_ae, p['Wd1'], p['bd1'], True)
    d2 = _dense(d1, p['Wd2'], p['bd2'], True)
    d3 = _dense(d2, p['Wd3'], p['bd3'], True)
    x_bar = _dense(d3, p['Wxb'], p['bxb'], False)

    ones_e = jnp.ones(src.shape[0], dtype=jnp.float32)
    deg = jnp.maximum(jax.ops.segment_sum(ones_e, dst, num_segments=n), 1.0)

    def gcn(h, w, active):
        s = _dense(h, w, None, False)
        out = jax.ops.segment_sum(s[src], dst, num_segments=n) / deg[:, None]
        return relu(out) if active else out

    def gt(h, wq, wk, wv, active):
        q_ = _dense(h, wq, None, False)
        k_ = _dense(h, wk, None, False)
        v_ = _dense(h, wv, None, False)
        sc = jnp.sum(q_[dst] * k_[src], axis=-1) / (q_.shape[-1] ** 0.5)
        m = jax.ops.segment_max(sc, dst, num_segments=n)
        ex = jnp.exp(sc - m[dst])
        den = jax.ops.segment_sum(ex, dst, num_segments=n)
        al = ex / (den[dst] + 1e-16)
        out = jax.ops.segment_sum(al[:, None] * v_[src], dst, num_segments=n)
        return relu(out) if active else out

    gcn_enc1 = gcn(x, p['Wg1'], True)
    gcn_enc2 = gcn((1 - A) * gcn_enc1 + A * e1, p['Wg2'], True)
    gcn_enc3 = gcn((1 - A) * gcn_enc2 + A * e2, p['Wg3'], True)
    z_gcn = gcn((1 - A) * gcn_enc3 + A * e3, p['Wg4'], False)

    g1 = gt(x, p['Wq1'], p['Wk1'], p['Wv1'], True)
    g2 = gt((1 - A) * g1 + A * e1, p['Wq2'], p['Wk2'], p['Wv2'], True)
    g3 = gt((1 - A) * g2 + A * e2, p['Wq3'], p['Wk3'], p['Wv3'], True)
    z_graph = gcn((1 - A) * g3 + A * e3, p['Wg4'], False)

    z_i = ALPHA * z_gcn + BETA * z_ae + p['gamma'] * z_graph
    z_l, edge_gcn_hat = _big(adj, z_i, z_gcn)

    gd1 = gcn(z_gcn, p['Wg5'], True)
    gd2 = gcn(gd1, p['Wg6'], True)
    gd3 = gcn(gd2, p['Wg7'], True)
    z_gcn_hat = gcn(gd3, p['Wg8'], True)

    td1 = gt(z_graph, p['Wq5'], p['Wk5'], p['Wv5'], True)
    td2 = gt(td1, p['Wq6'], p['Wk6'], p['Wv6'], True)
    td3 = gt(td2, p['Wq7'], p['Wk7'], p['Wv7'], True)
    z_graph_hat = gcn(td3, p['Wg8'], True)

    q = _tdist(z_l, p['cluster'])
    q1 = _tdist(z_ae, p['cluster'])

    return (x_bar, z_gcn_hat, z_graph_hat, edge_gcn_hat, z_ae, q, q1, z_l)
